# baseline (speedup 1.0000x reference)
import jax
import jax.numpy as jnp
from jax import lax
from jax.experimental import pallas as pl
from jax.experimental.pallas import tpu as pltpu
from jax.experimental.pallas import tpu_sc as plsc

B = 16384
F = 26
ROWS_PER_FIELD = 40000
NC = 2
NS = 16
NW = NC * NS
BPW = B // NW
LANES = 16
GPF = BPW // LANES


def _body(x_hbm, table_hbm, bias_hbm, out_hbm, x_v, idx_v, emb_v, out_v, bias_v, sem):
    wid = lax.axis_index("s") * NC + lax.axis_index("c")
    base = wid * BPW

    pltpu.sync_copy(x_hbm.at[:, pl.ds(base, BPW)], x_v)
    pltpu.sync_copy(bias_hbm, bias_v)

    def step(f, _):
        off = f * ROWS_PER_FIELD

        def bgrp(q, _):
            for g in range(8):
                col = q * 8 * LANES + g * LANES
                idx_v[pl.ds(f * BPW + col, LANES)] = x_v[f, pl.ds(col, LANES)] + off
            return 0

        lax.fori_loop(0, GPF // 8, bgrp, 0)
        pltpu.async_copy(
            table_hbm.at[0].at[idx_v.at[pl.ds(f * BPW, BPW)]],
            emb_v.at[pl.ds(f * BPW, BPW)],
            sem,
        )
        return 0

    lax.fori_loop(0, F, step, 0)
    pltpu.make_async_copy(table_hbm.at[0].at[pl.ds(0, F * BPW)], emb_v, sem).wait()

    bias_vec = bias_v[...]

    def red(s, _):
        col = s * LANES

        def radd(h, acc):
            f0 = h * 13
            for d in range(13):
                acc = acc + emb_v[pl.ds((f0 + d) * BPW + col, LANES)]
            return acc

        out_v[pl.ds(col, LANES)] = lax.fori_loop(0, 2, radd, bias_vec)
        return 0

    lax.fori_loop(0, GPF, red, 0)

    pltpu.sync_copy(out_v, out_hbm.at[pl.ds(base, BPW)])


def kernel(x, table, bias):
    xf = x.astype(jnp.int32).T
    bb = jnp.tile(bias.astype(jnp.float32), LANES)
    run = pl.kernel(
        _body,
        mesh=plsc.VectorSubcoreMesh(core_axis_name="c", subcore_axis_name="s"),
        out_type=jax.ShapeDtypeStruct((B,), jnp.float32),
        scratch_types=[
            pltpu.VMEM((F, BPW), jnp.int32),
            pltpu.VMEM((F * BPW,), jnp.int32),
            pltpu.VMEM((F * BPW,), jnp.float32),
            pltpu.VMEM((BPW,), jnp.float32),
            pltpu.VMEM((LANES,), jnp.float32),
            pltpu.SemaphoreType.DMA,
        ],
    )
    out = run(xf, table.T, bb)
    return out.reshape(B, 1)


# speedup vs baseline: 1.0030x; 1.0030x over previous
import jax
import jax.numpy as jnp
from jax import lax
from jax.experimental import pallas as pl
from jax.experimental.pallas import tpu as pltpu
from jax.experimental.pallas import tpu_sc as plsc

B = 16384
F = 26
ROWS_PER_FIELD = 40000
NC = 2
NS = 16
NW = NC * NS
BPW = B // NW
LANES = 16
GPF = BPW // LANES


def _body(x_hbm, table_hbm, bias_hbm, out_hbm, x_v, idx_v, emb_v, out_v, bias_v, sem):
    wid = lax.axis_index("s") * NC + lax.axis_index("c")
    base = wid * BPW

    pltpu.sync_copy(x_hbm.at[:, pl.ds(base, BPW)], x_v)
    pltpu.sync_copy(bias_hbm, bias_v)

    def step(f, _):
        off = f * ROWS_PER_FIELD
        for g in range(GPF):
            idx_v[pl.ds(f * BPW + g * LANES, LANES)] = (
                x_v[f, pl.ds(g * LANES, LANES)] + off
            )
        pltpu.async_copy(
            table_hbm.at[0].at[idx_v.at[pl.ds(f * BPW, BPW)]],
            emb_v.at[pl.ds(f * BPW, BPW)],
            sem,
        )
        return 0

    lax.fori_loop(0, F, step, 0)
    pltpu.make_async_copy(table_hbm.at[0].at[pl.ds(0, F * BPW)], emb_v, sem).wait()

    bias_vec = bias_v[...]

    def red(s, _):
        col = s * LANES
        acc = bias_vec
        for f in range(F):
            acc = acc + emb_v[pl.ds(f * BPW + col, LANES)]
        out_v[pl.ds(col, LANES)] = acc
        return 0

    lax.fori_loop(0, GPF, red, 0)

    pltpu.sync_copy(out_v, out_hbm.at[pl.ds(base, BPW)])


def kernel(x, table, bias):
    xf = x.astype(jnp.int32).T
    bb = jnp.tile(bias.astype(jnp.float32), LANES)
    run = pl.kernel(
        _body,
        mesh=plsc.VectorSubcoreMesh(core_axis_name="c", subcore_axis_name="s"),
        out_type=jax.ShapeDtypeStruct((B,), jnp.float32),
        scratch_types=[
            pltpu.VMEM((F, BPW), jnp.int32),
            pltpu.VMEM((F * BPW,), jnp.int32),
            pltpu.VMEM((F * BPW,), jnp.float32),
            pltpu.VMEM((BPW,), jnp.float32),
            pltpu.VMEM((LANES,), jnp.float32),
            pltpu.SemaphoreType.DMA,
        ],
    )
    out = run(xf, table.T, bb)
    return out.reshape(B, 1)
